# Initial kernel scaffold; baseline (speedup 1.0000x reference)
import jax
import jax.numpy as jnp
from jax.experimental import pallas as pl


def kernel(U, V, X):
    raise NotImplementedError("write your pallas kernel here")


# speedup vs baseline: 173.8081x; 173.8081x over previous
import jax
import jax.numpy as jnp
from jax import lax
from jax.experimental import pallas as pl
from jax.experimental.pallas import tpu as pltpu

_M, _N, _R = 512, 256, 32
_SIGMA0 = 1.0
_C = 1.345
_MU = 0.1
_ITERS = 2


def _alpha_const():
    c2 = _C * _C
    import math
    f1 = math.erf(math.sqrt(c2 / 2.0))
    f3 = f1 - math.sqrt(2.0 / math.pi) * math.sqrt(c2) * math.exp(-c2 / 2.0)
    return 0.5 * c2 * (1.0 - f1) + 0.5 * f3


_ALPHA_C = _alpha_const()


def _tril_ones(L):
    r = lax.broadcasted_iota(jnp.int32, (L, L), 0)
    c = lax.broadcasted_iota(jnp.int32, (L, L), 1)
    return jnp.where(r >= c, 1.0, 0.0).astype(jnp.float32)


def _phase(Y, Mf, B0, G_ref, I_ref, pos_ref, tgt_ref, psi_ref, w_ref,
           mm_xm_b, mm_xmT_w, gram_row, L, C):
    f32 = jnp.float32
    for r in range(_R):
        G_ref[r] = gram_row(r)
    eye_rs = jnp.where(
        lax.broadcasted_iota(jnp.int32, (_R, _R, 1), 0)
        == lax.broadcasted_iota(jnp.int32, (_R, _R, 1), 1), 1.0, 0.0).astype(f32)
    I_ref[...] = jnp.broadcast_to(eye_rs, (_R, _R, C))

    iota_s = lax.broadcasted_iota(jnp.int32, (1, _R, 1), 1)
    iota_a = lax.broadcasted_iota(jnp.int32, (_R, 1, 1), 0)

    def gj_body(p, carry):
        oh_s = jnp.where(iota_s == p, 1.0, 0.0).astype(f32)
        grow = G_ref[pl.ds(p, 1)]
        irow = I_ref[pl.ds(p, 1)]
        d = jnp.sum(grow * oh_s, axis=1, keepdims=True)
        inv_d = 1.0 / d
        grow_n = grow * inv_d
        irow_n = irow * inv_d
        Gall = G_ref[...]
        Iall = I_ref[...]
        cfac = jnp.sum(Gall * oh_s, axis=1, keepdims=True)
        is_p = iota_a == p
        G_ref[...] = jnp.where(is_p, grow_n, Gall - cfac * grow_n)
        I_ref[...] = jnp.where(is_p, irow_n, Iall - cfac * irow_n)
        return carry

    lax.fori_loop(0, _R, gj_body, 0)

    n_vec = jnp.sum(Mf, axis=0, keepdims=True)
    Ltri = _tril_ones(L)
    cm = jnp.round(jnp.dot(Ltri, Mf, preferred_element_type=f32))
    pos_ref[...] = cm - 1.0
    denom = jnp.sqrt(2.0 * n_vec * jnp.asarray(_ALPHA_C, f32))

    B = B0
    sigma = jnp.full((1, C), _SIGMA0, dtype=f32)
    CH = 8
    for _ in range(_ITERS):
        Rm = Y - mm_xm_b(B)
        s = Rm / sigma
        clip = jnp.abs(s) <= _C
        psi = jnp.where(clip, s, jnp.where(s >= 0, _C, -_C))
        norm = jnp.sqrt(jnp.sum(Mf * psi * psi, axis=0, keepdims=True))
        sigma = norm / denom
        s = Rm / sigma
        clip = jnp.abs(s) <= _C
        psi = jnp.where(clip, s, jnp.where(s >= 0, _C, -_C))
        psi_ref[...] = psi
        clip_f = jnp.where(clip, 1.0, 0.0).astype(f32)
        mc = Mf * clip_f
        cc = jnp.round(jnp.dot(Ltri, mc, preferred_element_type=f32))
        n1 = jnp.sum(mc, axis=0, keepdims=True)
        tgt_ref[...] = jnp.where(
            mc > 0, cc - 1.0,
            jnp.where(Mf > 0, n1 + (cm - cc) - 1.0, -7.0))

        def oh_body(k, carry):
            po = pos_ref[pl.ds(k * CH, CH), :]
            t_all = tgt_ref[...]
            p_all = psi_ref[...]
            cmpm = t_all[None, :, :] == po[:, None, :]
            w_ref[pl.ds(k * CH, CH), :] = jnp.sum(
                jnp.where(cmpm, p_all[None, :, :], 0.0), axis=1)
            return carry

        lax.fori_loop(0, L // CH, oh_body, 0)
        W = w_ref[...] * sigma * Mf
        t = mm_xmT_w(W)
        delta = jnp.sum(I_ref[...] * t[None, :, :], axis=1)
        B = B + _MU * delta
    return B


def _whole_kernel(U_ref, UT_ref, V_ref, X_ref, XT_ref, P_ref,
                  Gv, Iv, posv, tgtv, psiv, wv,
                  Gu, Iu, posu, tgtu, psiu, wu):
    f32 = jnp.float32
    U = U_ref[...]
    V0 = V_ref[...]
    X = X_ref[...]
    XT = XT_ref[...]
    UT = UT_ref[...]

    Mv = jnp.where(X != 0.0, 1.0, 0.0).astype(f32)

    def v_xm_b(B):
        return jnp.dot(U, B, preferred_element_type=f32)

    def v_xmT_w(W):
        return lax.dot_general(U, W, (((0,), (0,)), ((), ())),
                               preferred_element_type=f32)

    def v_gram_row(r):
        Ar = U * U[:, r:r + 1]
        return lax.dot_general(Ar, Mv, (((0,), (0,)), ((), ())),
                               preferred_element_type=f32)

    Vn = _phase(X, Mv, V0, Gv, Iv, posv, tgtv, psiv, wv,
                v_xm_b, v_xmT_w, v_gram_row, _M, _N)

    Mu = jnp.where(XT != 0.0, 1.0, 0.0).astype(f32)

    def u_xm_b(B):
        return lax.dot_general(Vn, B, (((0,), (0,)), ((), ())),
                               preferred_element_type=f32)

    def u_xmT_w(W):
        return jnp.dot(Vn, W, preferred_element_type=f32)

    def u_gram_row(r):
        Ar = Vn * Vn[r:r + 1, :]
        return jnp.dot(Ar, Mu, preferred_element_type=f32)

    Un = _phase(XT, Mu, UT, Gu, Iu, posu, tgtu, psiu, wu,
                u_xm_b, u_xmT_w, u_gram_row, _N, _M)

    P_ref[...] = lax.dot_general(Un, Vn, (((0,), (0,)), ((), ())),
                                 preferred_element_type=f32)


def kernel(U, V, X):
    f32 = jnp.float32
    scratch = [
        pltpu.VMEM((_R, _R, _N), f32), pltpu.VMEM((_R, _R, _N), f32),
        pltpu.VMEM((_M, _N), f32), pltpu.VMEM((_M, _N), f32),
        pltpu.VMEM((_M, _N), f32), pltpu.VMEM((_M, _N), f32),
        pltpu.VMEM((_R, _R, _M), f32), pltpu.VMEM((_R, _R, _M), f32),
        pltpu.VMEM((_N, _M), f32), pltpu.VMEM((_N, _M), f32),
        pltpu.VMEM((_N, _M), f32), pltpu.VMEM((_N, _M), f32),
    ]
    return pl.pallas_call(
        _whole_kernel,
        out_shape=jax.ShapeDtypeStruct((_M, _N), f32),
        scratch_shapes=scratch,
    )(U.astype(f32), U.T.astype(f32), V.astype(f32), X.astype(f32),
      X.T.astype(f32))
